# Initial kernel scaffold; baseline (speedup 1.0000x reference)
import jax
import jax.numpy as jnp
from jax.experimental import pallas as pl


def kernel(locations, confidences, dboxes, targets):
    raise NotImplementedError("write your pallas kernel here")


# speedup vs baseline: 13.9712x; 13.9712x over previous
import functools

import jax
import jax.numpy as jnp
from jax import lax
from jax.experimental import pallas as pl
from jax.experimental.pallas import tpu as pltpu

JACCARD_THRESH = 0.5
NEGPOS_RATIO = 3
VAR0, VAR1 = 0.1, 0.2


def _ssd_body(conf_ref, loc_ref, dbox_ref, tgt_ref, out_ref,
              lc_ref, np_ref, acc_ref, *, B, D, C, NOBJ):
    i = pl.program_id(0)

    @pl.when(i < B)
    def per_image():
        t = tgt_ref[0]
        tx1 = t[:, 0:1]
        ty1 = t[:, 1:2]
        tx2 = t[:, 2:3]
        ty2 = t[:, 3:4]

        db = dbox_ref[...]
        pcx = db[0:1, :]
        pcy = db[1:2, :]
        pw = db[2:3, :]
        ph = db[3:4, :]
        px1 = pcx - pw / 2.0
        py1 = pcy - ph / 2.0
        px2 = pcx + pw / 2.0
        py2 = pcy + ph / 2.0

        iw = jnp.maximum(jnp.minimum(tx2, px2) - jnp.maximum(tx1, px1), 0.0)
        ih = jnp.maximum(jnp.minimum(ty2, py2) - jnp.maximum(ty1, py1), 0.0)
        inter = iw * ih
        area_t = (tx2 - tx1) * (ty2 - ty1)
        area_p = (px2 - px1) * (py2 - py1)
        ov = inter / (area_t + area_p - inter)

        bto = jnp.max(ov, axis=0, keepdims=True)
        row_iota = lax.broadcasted_iota(jnp.int32, (NOBJ, D), 0)
        bti = jnp.min(jnp.where(ov == bto, row_iota, NOBJ),
                      axis=0, keepdims=True)

        bpo = jnp.max(ov, axis=1, keepdims=True)
        lane_iota = lax.broadcasted_iota(jnp.int32, (NOBJ, D), 1)
        bp = jnp.min(jnp.where(ov == bpo, lane_iota, D),
                     axis=1, keepdims=True)

        lane1 = lax.broadcasted_iota(jnp.int32, (1, D), 1)
        for j in range(NOBJ):
            m = lane1 == bp[j, 0]
            bto = jnp.where(m, 2.0, bto)
            bti = jnp.where(m, j, bti)

        mx1 = jnp.zeros((1, D), jnp.float32) + t[0, 0]
        my1 = jnp.zeros((1, D), jnp.float32) + t[0, 1]
        mx2 = jnp.zeros((1, D), jnp.float32) + t[0, 2]
        my2 = jnp.zeros((1, D), jnp.float32) + t[0, 3]
        cls = jnp.zeros((1, D), jnp.int32) + (t[0, 4].astype(jnp.int32) + 1)
        for j in range(1, NOBJ):
            m = bti == j
            mx1 = jnp.where(m, t[j, 0], mx1)
            my1 = jnp.where(m, t[j, 1], my1)
            mx2 = jnp.where(m, t[j, 2], mx2)
            my2 = jnp.where(m, t[j, 3], my2)
            cls = jnp.where(m, t[j, 4].astype(jnp.int32) + 1, cls)

        pos = bto >= JACCARD_THRESH
        cls = jnp.where(pos, cls, 0)
        posf = pos.astype(jnp.float32)

        gcx = ((mx1 + mx2) / 2.0 - pcx) / (VAR0 * pw)
        gcy = ((my1 + my2) / 2.0 - pcy) / (VAR0 * ph)
        gw = jnp.log((mx2 - mx1) / pw) / VAR1
        gh = jnp.log((my2 - my1) / ph) / VAR1

        lp = loc_ref[0]
        g = jnp.concatenate([gcx, gcy, gw, gh], axis=0)
        diff = lp - g
        ad = jnp.abs(diff)
        sl1 = jnp.where(ad < 1.0, 0.5 * diff * diff, ad - 0.5)
        sl1_i = jnp.sum(sl1 * posf)

        c = conf_ref[0]
        cmax = jnp.max(c, axis=0, keepdims=True)
        sh = c - cmax
        se = jnp.sum(jnp.exp(sh), axis=0, keepdims=True)
        lse = jnp.log(se)
        sel = sh[0:1, :]
        for k in range(1, C):
            sel = jnp.where(cls == k, sh[k:k + 1, :], sel)
        ce = lse - sel

        lc = jnp.where(pos, 0.0, ce)
        lc_ref[pl.ds(i, 1), :] = lc

        np_i = jnp.sum(posf)
        np_ref[pl.ds(i, 1), :] = jnp.full((1, 128), np_i, jnp.float32)

        cep_i = jnp.sum(jnp.where(pos, ce, 0.0))
        acc_ref[0] = jnp.where(i == 0, 0.0, acc_ref[0]) + sl1_i
        acc_ref[1] = jnp.where(i == 0, 0.0, acc_ref[1]) + cep_i
        acc_ref[2] = jnp.where(i == 0, 0.0, acc_ref[2]) + np_i

    @pl.when(i == B)
    def finalize():
        lc = lc_ref[...]
        lcb = lax.bitcast_convert_type(lc, jnp.int32)
        npv = np_ref[:, 0:1]
        k = jnp.minimum(npv * float(NEGPOS_RATIO), float(D))

        lo = jnp.full((B, 1), -1, jnp.int32)
        hi = jnp.full((B, 1), 0x7F800000, jnp.int32)

        def step(_, carry):
            lo, hi = carry
            mid = lo + lax.shift_right_logical(hi - lo, 1)
            cnt = jnp.sum((lcb > mid).astype(jnp.float32),
                          axis=1, keepdims=True)
            ge = cnt >= k
            return jnp.where(ge, mid, lo), jnp.where(ge, hi, mid)

        lo, hi = lax.fori_loop(0, 31, step, (lo, hi))
        thr = lax.bitcast_convert_type(hi, jnp.float32)

        gtf = (lc > thr).astype(jnp.float32)
        cnt_gt = jnp.sum(gtf, axis=1, keepdims=True)
        sum_gt = jnp.sum(lc * gtf, axis=1, keepdims=True)
        negv = sum_gt + thr * (k - cnt_gt)
        neg_total = jnp.sum(negv)

        n = acc_ref[2]
        out_ref[0] = acc_ref[0] / n
        out_ref[1] = (acc_ref[1] + neg_total) / n


def kernel(locations, confidences, dboxes, targets):
    B, D, _ = locations.shape
    C = confidences.shape[-1]
    NOBJ = targets.shape[1]

    conf_t = jnp.transpose(confidences, (0, 2, 1))
    loc_t = jnp.transpose(locations, (0, 2, 1))
    dbox_t = dboxes.T

    last = B - 1
    body = functools.partial(_ssd_body, B=B, D=D, C=C, NOBJ=NOBJ)
    out = pl.pallas_call(
        body,
        grid=(B + 1,),
        in_specs=[
            pl.BlockSpec((1, C, D), lambda i: (jnp.minimum(i, last), 0, 0)),
            pl.BlockSpec((1, 4, D), lambda i: (jnp.minimum(i, last), 0, 0)),
            pl.BlockSpec((4, D), lambda i: (0, 0)),
            pl.BlockSpec((1, NOBJ, 5), lambda i: (jnp.minimum(i, last), 0, 0)),
        ],
        out_specs=pl.BlockSpec(memory_space=pltpu.SMEM),
        out_shape=jax.ShapeDtypeStruct((2,), jnp.float32),
        scratch_shapes=[
            pltpu.VMEM((B, D), jnp.float32),
            pltpu.VMEM((B, 128), jnp.float32),
            pltpu.SMEM((3,), jnp.float32),
        ],
    )(conf_t, loc_t, dbox_t, targets)
    return out[0], out[1]


# speedup vs baseline: 35.5924x; 2.5476x over previous
import functools

import jax
import jax.numpy as jnp
from jax import lax
from jax.experimental import pallas as pl
from jax.experimental.pallas import tpu as pltpu

JACCARD_THRESH = 0.5
NEGPOS_RATIO = 3
VAR0, VAR1 = 0.1, 0.2
GRP = 8


def _ssd_body(conf_ref, loc_ref, dbox_ref, tgt_ref, out_ref,
              lc_ref, np_ref, acc_ref, *, B, D, C, NOBJ):
    g = pl.program_id(0)
    ngrp = B // GRP

    @pl.when(g < ngrp)
    def per_group():
        db = dbox_ref[...]
        pcx = db[0:1, :]
        pcy = db[1:2, :]
        pw = db[2:3, :]
        ph = db[3:4, :]
        px1 = pcx - pw / 2.0
        py1 = pcy - ph / 2.0
        px2 = pcx + pw / 2.0
        py2 = pcy + ph / 2.0
        area_p = (px2 - px1) * (py2 - py1)
        lane1 = lax.broadcasted_iota(jnp.int32, (1, D), 1)
        row_iota = lax.broadcasted_iota(jnp.int32, (NOBJ, D), 0)
        lane_iota = lax.broadcasted_iota(jnp.int32, (NOBJ, D), 1)

        bto_rows = []
        bti_rows = []
        for j in range(GRP):
            t = tgt_ref[j]
            tx1 = t[:, 0:1]
            ty1 = t[:, 1:2]
            tx2 = t[:, 2:3]
            ty2 = t[:, 3:4]

            iw = jnp.maximum(jnp.minimum(tx2, px2) - jnp.maximum(tx1, px1), 0.0)
            ih = jnp.maximum(jnp.minimum(ty2, py2) - jnp.maximum(ty1, py1), 0.0)
            inter = iw * ih
            area_t = (tx2 - tx1) * (ty2 - ty1)
            ov = inter / (area_t + area_p - inter)

            bto = jnp.max(ov, axis=0, keepdims=True)
            bti = jnp.min(jnp.where(ov == bto, row_iota, NOBJ),
                          axis=0, keepdims=True)
            bpo = jnp.max(ov, axis=1, keepdims=True)
            bp = jnp.min(jnp.where(ov == bpo, lane_iota, D),
                         axis=1, keepdims=True)
            for o in range(NOBJ):
                m = lane1 == bp[o, 0]
                bto = jnp.where(m, 2.0, bto)
                bti = jnp.where(m, o, bti)
            bto_rows.append(bto)
            bti_rows.append(bti)

        bto8 = jnp.concatenate(bto_rows, axis=0)
        bti8 = jnp.concatenate(bti_rows, axis=0)

        tg = tgt_ref[...]
        mx1 = jnp.zeros((GRP, D), jnp.float32) + tg[:, 0, 0:1]
        my1 = jnp.zeros((GRP, D), jnp.float32) + tg[:, 0, 1:2]
        mx2 = jnp.zeros((GRP, D), jnp.float32) + tg[:, 0, 2:3]
        my2 = jnp.zeros((GRP, D), jnp.float32) + tg[:, 0, 3:4]
        cls = jnp.zeros((GRP, D), jnp.int32) + (
            tg[:, 0, 4:5].astype(jnp.int32) + 1)
        for o in range(1, NOBJ):
            m = bti8 == o
            mx1 = jnp.where(m, tg[:, o, 0:1], mx1)
            my1 = jnp.where(m, tg[:, o, 1:2], my1)
            mx2 = jnp.where(m, tg[:, o, 2:3], mx2)
            my2 = jnp.where(m, tg[:, o, 3:4], my2)
            cls = jnp.where(m, tg[:, o, 4:5].astype(jnp.int32) + 1, cls)

        pos = bto8 >= JACCARD_THRESH
        cls = jnp.where(pos, cls, 0)
        posf = pos.astype(jnp.float32)

        gcx = ((mx1 + mx2) / 2.0 - pcx) / (VAR0 * pw)
        gcy = ((my1 + my2) / 2.0 - pcy) / (VAR0 * ph)
        gw = jnp.log((mx2 - mx1) / pw) / VAR1
        gh = jnp.log((my2 - my1) / ph) / VAR1

        sl1_g = 0.0
        for c, enc in ((0, gcx), (1, gcy), (2, gw), (3, gh)):
            diff = loc_ref[c] - enc
            ad = jnp.abs(diff)
            sl1 = jnp.where(ad < 1.0, 0.5 * diff * diff, ad - 0.5)
            sl1_g = sl1_g + jnp.sum(sl1 * posf)

        c3 = conf_ref[...]
        mx = c3[0]
        for k in range(1, C):
            mx = jnp.maximum(mx, c3[k])
        se = jnp.zeros((GRP, D), jnp.float32)
        sel = jnp.zeros((GRP, D), jnp.float32)
        for k in range(C):
            sh = c3[k] - mx
            se = se + jnp.exp(sh)
            sel = jnp.where(cls == k, sh, sel)
        ce = jnp.log(se) - sel

        lc_ref[pl.ds(g * GRP, GRP), :] = jnp.where(pos, 0.0, ce)
        npv = jnp.sum(posf, axis=1, keepdims=True)
        np_ref[pl.ds(g * GRP, GRP), :] = jnp.broadcast_to(npv, (GRP, 128))

        cep_g = jnp.sum(jnp.where(pos, ce, 0.0))
        acc_ref[0] = jnp.where(g == 0, 0.0, acc_ref[0]) + sl1_g
        acc_ref[1] = jnp.where(g == 0, 0.0, acc_ref[1]) + cep_g
        acc_ref[2] = jnp.where(g == 0, 0.0, acc_ref[2]) + jnp.sum(npv)

    @pl.when(g == ngrp)
    def finalize():
        lc = lc_ref[...]
        lcb = lax.bitcast_convert_type(lc, jnp.int32)
        npv = np_ref[:, 0:1]
        k = jnp.minimum(npv * float(NEGPOS_RATIO), float(D))

        lo = jnp.full((B, 1), -1, jnp.int32)
        hi = jnp.full((B, 1), 0x7F800000, jnp.int32)

        def step(_, carry):
            lo, hi = carry
            mid = lo + lax.shift_right_logical(hi - lo, 1)
            cnt = jnp.sum((lcb > mid).astype(jnp.float32),
                          axis=1, keepdims=True)
            ge = cnt >= k
            return jnp.where(ge, mid, lo), jnp.where(ge, hi, mid)

        lo, hi = lax.fori_loop(0, 31, step, (lo, hi))
        thr = lax.bitcast_convert_type(hi, jnp.float32)

        gtf = (lc > thr).astype(jnp.float32)
        cnt_gt = jnp.sum(gtf, axis=1, keepdims=True)
        sum_gt = jnp.sum(lc * gtf, axis=1, keepdims=True)
        neg_total = jnp.sum(sum_gt + thr * (k - cnt_gt))

        n = acc_ref[2]
        out_ref[0] = acc_ref[0] / n
        out_ref[1] = (acc_ref[1] + neg_total) / n


def kernel(locations, confidences, dboxes, targets):
    B, D, _ = locations.shape
    C = confidences.shape[-1]
    NOBJ = targets.shape[1]
    ngrp = B // GRP

    conf_t = jnp.transpose(confidences, (2, 0, 1))
    loc_t = jnp.transpose(locations, (2, 0, 1))
    dbox_t = dboxes.T

    last = ngrp - 1
    body = functools.partial(_ssd_body, B=B, D=D, C=C, NOBJ=NOBJ)
    out = pl.pallas_call(
        body,
        grid=(ngrp + 1,),
        in_specs=[
            pl.BlockSpec((C, GRP, D), lambda i: (0, jnp.minimum(i, last), 0)),
            pl.BlockSpec((4, GRP, D), lambda i: (0, jnp.minimum(i, last), 0)),
            pl.BlockSpec((4, D), lambda i: (0, 0)),
            pl.BlockSpec((GRP, NOBJ, 5),
                         lambda i: (jnp.minimum(i, last), 0, 0)),
        ],
        out_specs=pl.BlockSpec(memory_space=pltpu.SMEM),
        out_shape=jax.ShapeDtypeStruct((2,), jnp.float32),
        scratch_shapes=[
            pltpu.VMEM((B, D), jnp.float32),
            pltpu.VMEM((B, 128), jnp.float32),
            pltpu.SMEM((3,), jnp.float32),
        ],
    )(conf_t, loc_t, dbox_t, targets)
    return out[0], out[1]


# speedup vs baseline: 46.4615x; 1.3054x over previous
import functools

import jax
import jax.numpy as jnp
from jax import lax
from jax.experimental import pallas as pl
from jax.experimental.pallas import tpu as pltpu

JACCARD_THRESH = 0.5
NEGPOS_RATIO = 3
VAR0, VAR1 = 0.1, 0.2
GRP = 8


def _ssd_body(conf_ref, loc_ref, dbox_ref, tgt_ref, out_ref,
              lc_ref, np_ref, acc_ref, *, B, D, C, NOBJ):
    g = pl.program_id(0)
    ngrp = B // GRP

    @pl.when(g < ngrp)
    def per_group():
        db = dbox_ref[...]
        pcx = db[0:1, :]
        pcy = db[1:2, :]
        pw = db[2:3, :]
        ph = db[3:4, :]
        px1 = pcx - pw / 2.0
        py1 = pcy - ph / 2.0
        px2 = pcx + pw / 2.0
        py2 = pcy + ph / 2.0
        area_p = (px2 - px1) * (py2 - py1)
        lane1 = lax.broadcasted_iota(jnp.int32, (1, D), 1)
        row_iota = lax.broadcasted_iota(jnp.int32, (NOBJ, D), 0)
        lane_iota = lax.broadcasted_iota(jnp.int32, (NOBJ, D), 1)

        bto_rows = []
        bti_rows = []
        bp_list = []
        for j in range(GRP):
            t = tgt_ref[j]
            tx1 = t[:, 0:1]
            ty1 = t[:, 1:2]
            tx2 = t[:, 2:3]
            ty2 = t[:, 3:4]

            iw = jnp.maximum(jnp.minimum(tx2, px2) - jnp.maximum(tx1, px1), 0.0)
            ih = jnp.maximum(jnp.minimum(ty2, py2) - jnp.maximum(ty1, py1), 0.0)
            inter = iw * ih
            area_t = (tx2 - tx1) * (ty2 - ty1)
            ov = inter / (area_t + area_p - inter)

            bto = jnp.max(ov, axis=0, keepdims=True)
            bti = jnp.min(jnp.where(ov == bto, row_iota, NOBJ),
                          axis=0, keepdims=True)
            bpo = jnp.max(ov, axis=1, keepdims=True)
            bp = jnp.min(jnp.where(ov == bpo, lane_iota, D),
                         axis=1, keepdims=True)
            bto_rows.append(bto)
            bti_rows.append(bti)
            bp_list.append(bp)

        bto8 = jnp.concatenate(bto_rows, axis=0)
        bti8 = jnp.concatenate(bti_rows, axis=0)

        lane8 = lax.broadcasted_iota(jnp.int32, (GRP, D), 1)
        row8 = lax.broadcasted_iota(jnp.int32, (GRP, 1), 0)
        for o in range(NOBJ):
            col = jnp.full((GRP, 1), bp_list[0][o, 0], jnp.int32)
            for j in range(1, GRP):
                col = jnp.where(row8 == j, bp_list[j][o, 0], col)
            m = lane8 == col
            bto8 = jnp.where(m, 2.0, bto8)
            bti8 = jnp.where(m, o, bti8)

        tg = tgt_ref[...]
        mx1 = jnp.zeros((GRP, D), jnp.float32) + tg[:, 0, 0:1]
        my1 = jnp.zeros((GRP, D), jnp.float32) + tg[:, 0, 1:2]
        mx2 = jnp.zeros((GRP, D), jnp.float32) + tg[:, 0, 2:3]
        my2 = jnp.zeros((GRP, D), jnp.float32) + tg[:, 0, 3:4]
        cls = jnp.zeros((GRP, D), jnp.int32) + (
            tg[:, 0, 4:5].astype(jnp.int32) + 1)
        for o in range(1, NOBJ):
            m = bti8 == o
            mx1 = jnp.where(m, tg[:, o, 0:1], mx1)
            my1 = jnp.where(m, tg[:, o, 1:2], my1)
            mx2 = jnp.where(m, tg[:, o, 2:3], mx2)
            my2 = jnp.where(m, tg[:, o, 3:4], my2)
            cls = jnp.where(m, tg[:, o, 4:5].astype(jnp.int32) + 1, cls)

        pos = bto8 >= JACCARD_THRESH
        cls = jnp.where(pos, cls, 0)
        posf = pos.astype(jnp.float32)

        gcx = ((mx1 + mx2) / 2.0 - pcx) / (VAR0 * pw)
        gcy = ((my1 + my2) / 2.0 - pcy) / (VAR0 * ph)
        gw = jnp.log((mx2 - mx1) / pw) / VAR1
        gh = jnp.log((my2 - my1) / ph) / VAR1

        sl1_g = 0.0
        for c, enc in ((0, gcx), (1, gcy), (2, gw), (3, gh)):
            diff = loc_ref[c] - enc
            ad = jnp.abs(diff)
            sl1 = jnp.where(ad < 1.0, 0.5 * diff * diff, ad - 0.5)
            sl1_g = sl1_g + jnp.sum(sl1 * posf)

        c3 = conf_ref[...]
        mx = c3[0]
        for k in range(1, C):
            mx = jnp.maximum(mx, c3[k])
        se = jnp.zeros((GRP, D), jnp.float32)
        sel = jnp.zeros((GRP, D), jnp.float32)
        for k in range(C):
            sh = c3[k] - mx
            se = se + jnp.exp(sh)
            sel = jnp.where(cls == k, sh, sel)
        ce = jnp.log(se) - sel

        lc_ref[pl.ds(g * GRP, GRP), :] = jnp.where(pos, 0.0, ce)
        npv = jnp.sum(posf, axis=1, keepdims=True)
        np_ref[pl.ds(g * GRP, GRP), :] = jnp.broadcast_to(npv, (GRP, 128))

        cep_g = jnp.sum(jnp.where(pos, ce, 0.0))
        acc_ref[0] = jnp.where(g == 0, 0.0, acc_ref[0]) + sl1_g
        acc_ref[1] = jnp.where(g == 0, 0.0, acc_ref[1]) + cep_g
        acc_ref[2] = jnp.where(g == 0, 0.0, acc_ref[2]) + jnp.sum(npv)

    @pl.when(g == ngrp)
    def finalize():
        lc = lc_ref[...]
        lcb = lax.bitcast_convert_type(lc, jnp.int32)
        npv = np_ref[:, 0:1]
        k = jnp.minimum(npv * float(NEGPOS_RATIO), float(D))

        lo = jnp.full((B, 1), -1, jnp.int32)
        hi = jnp.full((B, 1), 0x7F800000, jnp.int32)

        def step(_, carry):
            lo, hi = carry
            mid = lo + lax.shift_right_logical(hi - lo, 1)
            cnt = jnp.sum((lcb > mid).astype(jnp.float32),
                          axis=1, keepdims=True)
            ge = cnt >= k
            return jnp.where(ge, mid, lo), jnp.where(ge, hi, mid)

        lo, hi = lax.fori_loop(0, 31, step, (lo, hi))
        thr = lax.bitcast_convert_type(hi, jnp.float32)

        gtf = (lc > thr).astype(jnp.float32)
        cnt_gt = jnp.sum(gtf, axis=1, keepdims=True)
        sum_gt = jnp.sum(lc * gtf, axis=1, keepdims=True)
        neg_total = jnp.sum(sum_gt + thr * (k - cnt_gt))

        n = acc_ref[2]
        out_ref[0] = acc_ref[0] / n
        out_ref[1] = (acc_ref[1] + neg_total) / n


def kernel(locations, confidences, dboxes, targets):
    B, D, _ = locations.shape
    C = confidences.shape[-1]
    NOBJ = targets.shape[1]
    ngrp = B // GRP

    conf_t = jnp.transpose(confidences, (2, 0, 1))
    loc_t = jnp.transpose(locations, (2, 0, 1))
    dbox_t = dboxes.T

    last = ngrp - 1
    body = functools.partial(_ssd_body, B=B, D=D, C=C, NOBJ=NOBJ)
    out = pl.pallas_call(
        body,
        grid=(ngrp + 1,),
        in_specs=[
            pl.BlockSpec((C, GRP, D), lambda i: (0, jnp.minimum(i, last), 0)),
            pl.BlockSpec((4, GRP, D), lambda i: (0, jnp.minimum(i, last), 0)),
            pl.BlockSpec((4, D), lambda i: (0, 0)),
            pl.BlockSpec((GRP, NOBJ, 5),
                         lambda i: (jnp.minimum(i, last), 0, 0)),
        ],
        out_specs=pl.BlockSpec(memory_space=pltpu.SMEM),
        out_shape=jax.ShapeDtypeStruct((2,), jnp.float32),
        scratch_shapes=[
            pltpu.VMEM((B, D), jnp.float32),
            pltpu.VMEM((B, 128), jnp.float32),
            pltpu.SMEM((3,), jnp.float32),
        ],
    )(conf_t, loc_t, dbox_t, targets)
    return out[0], out[1]
